# baseline (speedup 1.0000x reference)
import jax
import jax.numpy as jnp
from jax import lax
from jax.experimental import pallas as pl
from jax.experimental.pallas import tpu as pltpu
from jax.experimental.pallas import tpu_sc as plsc

_ROWS = 512
_COLS = 512
_KS = 5
_WI = _KS // 2

_NC = 2
_NS = 16
_NW = _NC * _NS
_RPW = _ROWS // _NW
_L = 16
_NCHUNK = _COLS // _L
_XROWS = _RPW + 2 * _WI
_CPAD = 8
_XCOLS = _COLS + 2 * _CPAD


def _sc_body(x_hbm, w_hbm, o_hbm, xbuf, wbuf, obuf):
    cid = lax.axis_index("c")
    sid = lax.axis_index("s")
    wid = sid * _NC + cid
    row0 = wid * _RPW

    zero = jnp.zeros((_L,), jnp.float32)
    for t in range(_XROWS):
        xbuf[t, pl.ds(0, _L)] = zero
        xbuf[t, pl.ds(_XCOLS - _L, _L)] = zero

    pltpu.sync_copy(
        x_hbm.at[pl.ds(row0, _XROWS), :],
        xbuf.at[pl.ds(0, _XROWS), pl.ds(_CPAD, _COLS)])
    pltpu.sync_copy(w_hbm, wbuf)

    wv = [wbuf[i, :] for i in range(_KS * _KS)]

    @plsc.parallel_loop(0, _RPW, step=1, unroll=1)
    def _(r):
        for c in range(_NCHUNK):
            terms = []
            for a in range(_KS):
                t = r + 2 * _WI - a
                for b in range(_KS):
                    off = c * _L + _CPAD + _WI - b
                    chunk = xbuf[t, pl.ds(off, _L)]
                    terms.append(wv[a * _KS + b] * chunk)
            while len(terms) > 1:
                nxt = [terms[j] + terms[j + 1]
                       for j in range(0, len(terms) - 1, 2)]
                if len(terms) % 2:
                    nxt.append(terms[-1])
                terms = nxt
            obuf[r, pl.ds(c * _L, _L)] = terms[0]

    pltpu.sync_copy(obuf, o_hbm.at[pl.ds(row0, _RPW), :])


@jax.jit
def _splat_sc(x, wvec):
    xp = jnp.zeros((_ROWS + 2 * _WI, _COLS), jnp.float32)
    xp = lax.dynamic_update_slice(xp, x, (_WI, 0))
    mesh = plsc.VectorSubcoreMesh(
        core_axis_name="c", subcore_axis_name="s",
        num_cores=_NC, num_subcores=_NS)
    return pl.kernel(
        _sc_body,
        out_type=jax.ShapeDtypeStruct((_ROWS, _COLS), jnp.float32),
        mesh=mesh,
        scratch_types=[
            pltpu.VMEM((_XROWS, _XCOLS), jnp.float32),
            pltpu.VMEM((_KS * _KS, _L), jnp.float32),
            pltpu.VMEM((_RPW, _COLS), jnp.float32),
        ],
        compiler_params=pltpu.CompilerParams(use_tc_tiling_on_sc=False),
    )(xp, wvec)


def kernel(input, kernel):
    wvec = jnp.tile(kernel.reshape(_KS * _KS, 1), (1, _L))
    return _splat_sc(input, wvec)


# speedup vs baseline: 1.1988x; 1.1988x over previous
import jax
import jax.numpy as jnp
from jax import lax
from jax.experimental import pallas as pl
from jax.experimental.pallas import tpu as pltpu
from jax.experimental.pallas import tpu_sc as plsc

_ROWS = 512
_COLS = 512
_KS = 5
_WI = _KS // 2

_NC = 2
_NS = 16
_NW = _NC * _NS
_RPW = _ROWS // _NW
_L = 16
_NCHUNK = _COLS // _L
_XROWS = _RPW + 2 * _WI
_CPAD = 8
_XCOLS = _COLS + 2 * _CPAD


def _sc_body(x_hbm, w_hbm, o_hbm, xbuf, wbuf, obuf):
    cid = lax.axis_index("c")
    sid = lax.axis_index("s")
    wid = sid * _NC + cid
    row0 = wid * _RPW

    zero = jnp.zeros((_L,), jnp.float32)
    for t in range(_XROWS):
        xbuf[t, pl.ds(0, _L)] = zero
        xbuf[t, pl.ds(_XCOLS - _L, _L)] = zero

    pltpu.sync_copy(
        x_hbm.at[pl.ds(row0, _XROWS), :],
        xbuf.at[pl.ds(0, _XROWS), pl.ds(_CPAD, _COLS)])
    pltpu.sync_copy(w_hbm, wbuf)

    ws = [wbuf[i, :][0] for i in range(_KS * _KS)]

    @plsc.parallel_loop(0, _RPW * _NCHUNK, step=1, unroll=4)
    def _(i):
        r = i // _NCHUNK
        c = i % _NCHUNK
        terms = []
        for a in range(_KS):
            t = r + 2 * _WI - a
            for b in range(_KS):
                off = c * _L + _CPAD + _WI - b
                chunk = xbuf[t, pl.ds(off, _L)]
                terms.append(ws[a * _KS + b] * chunk)
        while len(terms) > 1:
            nxt = [terms[j] + terms[j + 1] for j in range(0, len(terms) - 1, 2)]
            if len(terms) % 2:
                nxt.append(terms[-1])
            terms = nxt
        obuf[r, pl.ds(c * _L, _L)] = terms[0]

    pltpu.sync_copy(obuf, o_hbm.at[pl.ds(row0, _RPW), :])


@jax.jit
def _splat_sc(x, wvec):
    xp = jnp.zeros((_ROWS + 2 * _WI, _COLS), jnp.float32)
    xp = lax.dynamic_update_slice(xp, x, (_WI, 0))
    mesh = plsc.VectorSubcoreMesh(
        core_axis_name="c", subcore_axis_name="s",
        num_cores=_NC, num_subcores=_NS)
    return pl.kernel(
        _sc_body,
        out_type=jax.ShapeDtypeStruct((_ROWS, _COLS), jnp.float32),
        mesh=mesh,
        scratch_types=[
            pltpu.VMEM((_XROWS, _XCOLS), jnp.float32),
            pltpu.VMEM((_KS * _KS, _L), jnp.float32),
            pltpu.VMEM((_RPW, _COLS), jnp.float32),
        ],
        compiler_params=pltpu.CompilerParams(use_tc_tiling_on_sc=False),
    )(xp, wvec)


def kernel(input, kernel):
    wvec = jnp.tile(kernel.reshape(_KS * _KS, 1), (1, _L))
    return _splat_sc(input, wvec)


# speedup vs baseline: 1.4761x; 1.2314x over previous
import jax
import jax.numpy as jnp
from jax import lax
from jax.experimental import pallas as pl
from jax.experimental.pallas import tpu as pltpu
from jax.experimental.pallas import tpu_sc as plsc

_ROWS = 512
_COLS = 512
_KS = 5
_WI = _KS // 2

_NC = 2
_NS = 16
_NW = _NC * _NS
_L = 16
_NCHUNK = _COLS // _L

_SC_ROWS = 64
_TC_ROWS = _ROWS - _SC_ROWS
_RPW = _SC_ROWS // _NW
_XROWS = _RPW + 2 * _WI
_CPAD = 8
_XCOLS = _COLS + 2 * _CPAD


def _sc_body(x_hbm, w_hbm, o_hbm, xbuf, wbuf, obuf):
    cid = lax.axis_index("c")
    sid = lax.axis_index("s")
    wid = sid * _NC + cid
    row0 = _TC_ROWS + wid * _RPW

    zero = jnp.zeros((_L,), jnp.float32)
    for t in range(_XROWS):
        xbuf[t, pl.ds(0, _L)] = zero
        xbuf[t, pl.ds(_XCOLS - _L, _L)] = zero

    pltpu.sync_copy(
        x_hbm.at[pl.ds(row0, _XROWS), :],
        xbuf.at[pl.ds(0, _XROWS), pl.ds(_CPAD, _COLS)])
    pltpu.sync_copy(w_hbm, wbuf)

    ws = [wbuf[i, :][0] for i in range(_KS * _KS)]

    @plsc.parallel_loop(0, _RPW * _NCHUNK, step=1, unroll=4)
    def _(i):
        r = i // _NCHUNK
        c = i % _NCHUNK
        terms = []
        for a in range(_KS):
            t = r + 2 * _WI - a
            for b in range(_KS):
                off = c * _L + _CPAD + _WI - b
                chunk = xbuf[t, pl.ds(off, _L)]
                terms.append(ws[a * _KS + b] * chunk)
        while len(terms) > 1:
            nxt = [terms[j] + terms[j + 1] for j in range(0, len(terms) - 1, 2)]
            if len(terms) % 2:
                nxt.append(terms[-1])
            terms = nxt
        obuf[r, pl.ds(c * _L, _L)] = terms[0]

    pltpu.sync_copy(obuf, o_hbm.at[pl.ds(wid * _RPW, _RPW), :])


_TCB = _TC_ROWS + 8


def _tc_body(kw_ref, x_ref, o_ref, r_ref):
    nr = _TC_ROWS
    x = x_ref[...]
    ras = [None] * _KS
    for v in range(_KS):
        d = v - _WI
        if d < 0:
            sv = jnp.concatenate(
                [jnp.zeros((_TCB, -d), jnp.float32),
                 x[:, :_COLS + d]], axis=1)
        elif d > 0:
            sv = jnp.concatenate(
                [x[:, d:], jnp.zeros((_TCB, d), jnp.float32)], axis=1)
        else:
            sv = x
        b = 2 * _WI - v
        for a in range(_KS):
            term = kw_ref[a, b] * sv
            ras[a] = term if ras[a] is None else ras[a] + term
    for a in range(_KS):
        r_ref[a, :, :] = ras[a]
    acc = None
    for a in range(_KS):
        u = 2 * _WI - a
        term = r_ref[a, u:u + nr, :]
        acc = term if acc is None else acc + term
    o_ref[...] = acc


@jax.jit
def _splat_hybrid(x, kw, wvec):
    xp = jnp.zeros((_ROWS + 2 * _WI, _COLS), jnp.float32)
    xp = lax.dynamic_update_slice(xp, x, (_WI, 0))

    mesh = plsc.VectorSubcoreMesh(
        core_axis_name="c", subcore_axis_name="s",
        num_cores=_NC, num_subcores=_NS)
    out_sc = pl.kernel(
        _sc_body,
        out_type=jax.ShapeDtypeStruct((_SC_ROWS, _COLS), jnp.float32),
        mesh=mesh,
        scratch_types=[
            pltpu.VMEM((_XROWS, _XCOLS), jnp.float32),
            pltpu.VMEM((_KS * _KS, _L), jnp.float32),
            pltpu.VMEM((_RPW, _COLS), jnp.float32),
        ],
        compiler_params=pltpu.CompilerParams(use_tc_tiling_on_sc=False),
    )(xp, wvec)

    out_tc = pl.pallas_call(
        _tc_body,
        out_shape=jax.ShapeDtypeStruct((_TC_ROWS, _COLS), jnp.float32),
        grid=(1,),
        in_specs=[
            pl.BlockSpec(memory_space=pltpu.SMEM),
            pl.BlockSpec((_TCB, _COLS), lambda i: (0, 0)),
        ],
        out_specs=pl.BlockSpec((_TC_ROWS, _COLS), lambda i: (0, 0)),
        scratch_shapes=[
            pltpu.VMEM((_KS, _TCB, _COLS), jnp.float32),
        ],
    )(kw, xp)

    return jnp.concatenate([out_tc, out_sc], axis=0)


def kernel(input, kernel):
    wvec = jnp.tile(kernel.reshape(_KS * _KS, 1), (1, _L))
    return _splat_hybrid(input, kernel, wvec)


# speedup vs baseline: 6.9633x; 4.7172x over previous
import jax
import jax.numpy as jnp
from jax.experimental import pallas as pl
from jax.experimental.pallas import tpu as pltpu

_ROWS = 512
_COLS = 512
_KS = 5
_WI = _KS // 2


def _splat_body(kw_ref, x_ref, o_ref, r_ref):
    x = x_ref[...]
    ras = [None] * _KS
    for v in range(_KS):
        d = v - _WI
        if d < 0:
            sv = jnp.concatenate(
                [jnp.zeros((_ROWS, -d), jnp.float32), x[:, :_COLS + d]], axis=1)
        elif d > 0:
            sv = jnp.concatenate(
                [x[:, d:], jnp.zeros((_ROWS, d), jnp.float32)], axis=1)
        else:
            sv = x
        b = 2 * _WI - v
        for a in range(_KS):
            term = kw_ref[a, b] * sv
            ras[a] = term if ras[a] is None else ras[a] + term
    for a in range(_KS):
        r_ref[a, :_WI, :] = jnp.zeros((_WI, _COLS), jnp.float32)
        r_ref[a, _WI:_WI + _ROWS, :] = ras[a]
        r_ref[a, _WI + _ROWS:, :] = jnp.zeros((_WI, _COLS), jnp.float32)
    acc = None
    for a in range(_KS):
        u = 2 * _WI - a
        term = r_ref[a, u:u + _ROWS, :]
        acc = term if acc is None else acc + term
    o_ref[...] = acc


def kernel(input, kernel):
    pad_rows = _ROWS + 2 * _WI
    return pl.pallas_call(
        _splat_body,
        out_shape=jax.ShapeDtypeStruct((_ROWS, _COLS), input.dtype),
        in_specs=[
            pl.BlockSpec(memory_space=pltpu.SMEM),
            pl.BlockSpec((_ROWS, _COLS), lambda: (0, 0)),
        ],
        out_specs=pl.BlockSpec((_ROWS, _COLS), lambda: (0, 0)),
        scratch_shapes=[
            pltpu.VMEM((_KS, pad_rows, _COLS), jnp.float32),
        ],
    )(kernel, input)
